# Initial kernel scaffold; baseline (speedup 1.0000x reference)
import jax
import jax.numpy as jnp
from jax.experimental import pallas as pl


def kernel(token_ids, W):
    raise NotImplementedError("write your pallas kernel here")


# speedup vs baseline: 1.1023x; 1.1023x over previous
import functools

import jax
import jax.numpy as jnp
from jax import lax
from jax.experimental import pallas as pl
from jax.experimental.pallas import tpu as pltpu
from jax.experimental.pallas import tpu_sc as plsc

_NUM_CORES = 2
_NUM_SUBCORES = 16
_NUM_WORKERS = _NUM_CORES * _NUM_SUBCORES

_CHUNK = 1600


@functools.lru_cache(maxsize=None)
def _make_gather(n_total: int, d: int, chunk: int):
    n_per_w = n_total // _NUM_WORKERS
    steps = n_per_w // chunk
    mesh = plsc.VectorSubcoreMesh(core_axis_name="c", subcore_axis_name="s")

    @functools.partial(
        pl.kernel,
        out_type=jax.ShapeDtypeStruct((n_total, d), jnp.float32),
        mesh=mesh,
        compiler_params=pltpu.CompilerParams(use_tc_tiling_on_sc=False),
        scratch_types=[
            pltpu.VMEM((chunk,), jnp.int32),
            pltpu.VMEM((chunk, d), jnp.float32),
            pltpu.SemaphoreType.DMA,
        ],
    )
    def gather(idx_hbm, w_hbm, out_hbm, idx_v, rows_v, sem):
        wid = lax.axis_index("s") * _NUM_CORES + lax.axis_index("c")
        base = wid * n_per_w

        def step(i, carry):
            off = pl.multiple_of(base + i * chunk, 8)
            pltpu.sync_copy(idx_hbm.at[pl.ds(off, chunk)], idx_v)
            pltpu.async_copy(w_hbm.at[idx_v], rows_v, sem).wait()
            pltpu.sync_copy(rows_v, out_hbm.at[pl.ds(off, chunk)])
            return carry

        lax.fori_loop(0, steps, step, 0)

    return gather


def kernel(token_ids, W):
    batch, hist = token_ids.shape
    _, d = W.shape
    n_total = batch * hist
    idx = token_ids.reshape(n_total).astype(jnp.int32)
    out = _make_gather(n_total, d, _CHUNK)(idx, W)
    return out.reshape(batch, hist, d)


# speedup vs baseline: 1.1131x; 1.0098x over previous
import functools

import jax
import jax.numpy as jnp
from jax import lax
from jax.experimental import pallas as pl
from jax.experimental.pallas import tpu as pltpu
from jax.experimental.pallas import tpu_sc as plsc

_NUM_CORES = 2
_NUM_SUBCORES = 16
_NUM_WORKERS = _NUM_CORES * _NUM_SUBCORES

_CHUNK = 1600
_NBUF = 2


@functools.lru_cache(maxsize=None)
def _make_gather(n_total: int, d: int, chunk: int):
    n_per_w = n_total // _NUM_WORKERS
    steps = n_per_w // chunk
    assert steps * chunk == n_per_w and steps >= _NBUF
    mesh = plsc.VectorSubcoreMesh(core_axis_name="c", subcore_axis_name="s")

    scratch = (
        [pltpu.VMEM((chunk,), jnp.int32) for _ in range(_NBUF)]
        + [pltpu.VMEM((chunk, d), jnp.float32) for _ in range(_NBUF)]
        + [pltpu.SemaphoreType.DMA for _ in range(3 * _NBUF)]
    )

    @functools.partial(
        pl.kernel,
        out_type=jax.ShapeDtypeStruct((n_total, d), jnp.float32),
        mesh=mesh,
        compiler_params=pltpu.CompilerParams(use_tc_tiling_on_sc=False),
        scratch_types=scratch,
    )
    def gather(idx_hbm, w_hbm, out_hbm, *refs):
        ibuf = refs[0:_NBUF]
        rbuf = refs[_NBUF:2 * _NBUF]
        isem = refs[2 * _NBUF:3 * _NBUF]
        gsem = refs[3 * _NBUF:4 * _NBUF]
        osem = refs[4 * _NBUF:5 * _NBUF]

        wid = lax.axis_index("s") * _NUM_CORES + lax.axis_index("c")
        base = wid * n_per_w

        def off(g):
            return pl.multiple_of(base + g * chunk, 8)

        idx_h = [None] * steps
        g_h = [None] * steps
        o_h = [None] * steps

        for g in range(_NBUF):
            idx_h[g] = pltpu.async_copy(
                idx_hbm.at[pl.ds(off(g), chunk)], ibuf[g], isem[g])

        for g in range(steps):
            b = g % _NBUF
            p = (g - 1) % _NBUF
            if g >= _NBUF:
                o_h[g - _NBUF].wait()
            idx_h[g].wait()
            g_h[g] = pltpu.async_copy(w_hbm.at[ibuf[b]], rbuf[b], gsem[b])
            if g >= 1:
                g_h[g - 1].wait()
                o_h[g - 1] = pltpu.async_copy(
                    rbuf[p], out_hbm.at[pl.ds(off(g - 1), chunk)], osem[p])
                if g + 1 < steps:
                    idx_h[g + 1] = pltpu.async_copy(
                        idx_hbm.at[pl.ds(off(g + 1), chunk)], ibuf[p], isem[p])

        last = steps - 1
        g_h[last].wait()
        o_h[last] = pltpu.async_copy(
            rbuf[last % _NBUF], out_hbm.at[pl.ds(off(last), chunk)],
            osem[last % _NBUF])
        o_h[last - 1].wait()
        o_h[last].wait()

    return gather


def kernel(token_ids, W):
    batch, hist = token_ids.shape
    _, d = W.shape
    n_total = batch * hist
    idx = token_ids.reshape(n_total).astype(jnp.int32)
    out = _make_gather(n_total, d, _CHUNK)(idx, W)
    return out.reshape(batch, hist, d)
